# baseline (speedup 1.0000x reference)
import functools

import jax
import jax.numpy as jnp
import numpy as np
from jax import lax
from jax.experimental import pallas as pl
from jax.experimental.pallas import tpu as pltpu
from jax.experimental.pallas import tpu_sc as plsc

K, T, D, H = 128, 64, 128, 256
TEMP = 1.0
EPS = 1e-12
LOG2 = float(np.log(2.0))

KB = 64
GRID = K // KB

NEG_INF = np.float32(-np.inf)

_GUMBEL = np.asarray(
    jax.random.gumbel(jax.random.key(42), (K, T * T), jnp.float32)
)

_DIAGMASK = np.where(
    (np.arange(T * T) // T) == (np.arange(T * T) % T), -np.inf, 0.0
).astype(np.float32)[None, :]

_LANEIDX = np.arange(T * T, dtype=np.int32)[None, :]


def _body_a(emb_ref, gum_ref, dmask_ref, lane_ref,
            idx1_ref, idx2_ref, g1_ref, g2_ref, logv_ref):
    diag2 = (lax.broadcasted_iota(jnp.int32, (T, T), 0)
             == lax.broadcasted_iota(jnp.int32, (T, T), 1))

    grams, ncols, nrows = [], [], []
    for i in range(KB):
        e = emb_ref[i]
        g = lax.dot_general(e, e, (((1,), (1,)), ((), ())),
                            preferred_element_type=jnp.float32,
                            precision=lax.Precision.HIGHEST)
        dv = jnp.where(diag2, g, 0.0)
        grams.append(g[None])
        ncols.append(jnp.sum(dv, axis=1, keepdims=True)[None])
        nrows.append(jnp.sum(dv, axis=0, keepdims=True)[None])

    g3 = jnp.concatenate(grams, axis=0)
    ncol3 = jnp.concatenate(ncols, axis=0)
    nrow3 = jnp.concatenate(nrows, axis=0)
    sq3 = jnp.maximum(ncol3 + nrow3 - 2.0 * g3, 0.0)
    sq2 = sq3.reshape(KB, T * T)
    lane = lane_ref[:]
    dist2 = jnp.sqrt(sq2 + EPS)
    logits2 = dmask_ref[:] - dist2 / TEMP
    scores2 = logits2 + gum_ref[:]

    m = jnp.max(scores2, axis=1, keepdims=True)
    samp = jnp.min(jnp.where(scores2 == m, lane, T * T),
                   axis=1, keepdims=True)
    i1s = samp // T
    i2s = samp % T
    chosen = jnp.sum(jnp.where(lane == samp, logits2, 0.0),
                     axis=1, keepdims=True)
    ml = jnp.max(logits2, axis=1, keepdims=True)
    s = jnp.sum(jnp.exp(logits2 - ml), axis=1, keepdims=True)
    lse = ml + jnp.log(s)
    idx1_ref[:] = i1s
    idx2_ref[:] = i2s
    krows = lax.broadcasted_iota(jnp.int32, (KB, 1), 0)
    base = (pl.program_id(0) * KB + krows) * T
    g1_ref[:] = base + i1s
    g2_ref[:] = base + i2s
    logv_ref[:] = chosen + LOG2 - lse


_SC_INFO = plsc.get_sparse_core_info()
_NW = _SC_INFO.num_cores * _SC_INFO.num_subcores
_B = 2 * K
_BPW = _B // _NW


@functools.partial(
    pl.kernel,
    out_type=jax.ShapeDtypeStruct((_B, D), jnp.float32),
    mesh=plsc.VectorSubcoreMesh(core_axis_name="c", subcore_axis_name="s"),
    scratch_types=[
        pltpu.VMEM((_BPW,), jnp.int32),
        pltpu.VMEM((_BPW, D), jnp.float32),
        pltpu.SemaphoreType.DMA,
    ],
)
def _sc_gather(table_hbm, idx_hbm, out_hbm, idx_v, rows_v, sem):
    wid = lax.axis_index("s") * _SC_INFO.num_cores + lax.axis_index("c")
    base = wid * _BPW
    pltpu.sync_copy(idx_hbm.at[pl.ds(base, _BPW)], idx_v)
    pltpu.async_copy(table_hbm.at[idx_v], rows_v, sem).wait()
    pltpu.sync_copy(rows_v, out_hbm.at[pl.ds(base, _BPW)])


def _body_b(rows_ref, w1_ref, b1_ref, w2_ref, b2_ref,
            br1_ref, br2_ref, out_ref):
    c1 = rows_ref[0:K, :]
    c2 = rows_ref[K:2 * K, :]
    cat = jnp.concatenate([c1, c2], axis=1)
    h = lax.dot_general(cat, w1_ref[:], (((1,), (0,)), ((), ())),
                        preferred_element_type=jnp.float32) + b1_ref[:]
    h = jnp.maximum(h, 0.0)
    out = lax.dot_general(h, w2_ref[:], (((1,), (0,)), ((), ())),
                          preferred_element_type=jnp.float32) + b2_ref[:]
    out_ref[:] = out
    br1_ref[:] = jnp.sqrt(jnp.sum((c1 - out) ** 2, axis=1, keepdims=True)
                          + EPS)
    br2_ref[:] = jnp.sqrt(jnp.sum((c2 - out) ** 2, axis=1, keepdims=True)
                          + EPS)


def kernel(N, leaf_counts_Kxt, embeddings_KxtxD, W1, b1, W2, b2):
    gum = jnp.asarray(_GUMBEL)
    idx1, idx2, g1, g2, logv = pl.pallas_call(
        _body_a,
        grid=(GRID,),
        in_specs=[
            pl.BlockSpec((KB, T, D), lambda i: (i, 0, 0)),
            pl.BlockSpec((KB, T * T), lambda i: (i, 0)),
            pl.BlockSpec((1, T * T), lambda i: (0, 0)),
            pl.BlockSpec((1, T * T), lambda i: (0, 0)),
        ],
        out_specs=[
            pl.BlockSpec((KB, 1), lambda i: (i, 0)),
            pl.BlockSpec((KB, 1), lambda i: (i, 0)),
            pl.BlockSpec((KB, 1), lambda i: (i, 0)),
            pl.BlockSpec((KB, 1), lambda i: (i, 0)),
            pl.BlockSpec((KB, 1), lambda i: (i, 0)),
        ],
        out_shape=[
            jax.ShapeDtypeStruct((K, 1), jnp.int32),
            jax.ShapeDtypeStruct((K, 1), jnp.int32),
            jax.ShapeDtypeStruct((K, 1), jnp.int32),
            jax.ShapeDtypeStruct((K, 1), jnp.int32),
            jax.ShapeDtypeStruct((K, 1), jnp.float32),
        ],
    )(embeddings_KxtxD, gum, jnp.asarray(_DIAGMASK), jnp.asarray(_LANEIDX))

    gidx = jnp.concatenate([g1[:, 0], g2[:, 0]], axis=0)
    table = embeddings_KxtxD.reshape(K * T, D)
    rows = _sc_gather(table, gidx)

    br1, br2, emb_out = pl.pallas_call(
        _body_b,
        out_shape=[
            jax.ShapeDtypeStruct((K, 1), jnp.float32),
            jax.ShapeDtypeStruct((K, 1), jnp.float32),
            jax.ShapeDtypeStruct((K, D), jnp.float32),
        ],
    )(rows, W1, b1.reshape(1, H), W2, b2.reshape(1, D))
    return (idx1, idx2, br1, br2, emb_out[:, None, :], logv)
